# baseline (speedup 1.0000x reference)
import functools
import math

import jax
import jax.numpy as jnp
import numpy as np
from jax import lax
from jax.experimental import pallas as pl
from jax.experimental.pallas import tpu as pltpu
from jax.experimental.pallas import tpu_sc as plsc

D_MODEL = 1024
SEQ = 4096
BATCH = 4
LANES = 16

NW = 32
ROWS_PER_W = BATCH * SEQ // NW
CHUNK = 32
NCHUNK = ROWS_PER_W // CHUNK
VPR = D_MODEL // LANES
SEQ_PER_W = SEQ // ROWS_PER_W


def _pe_table():
    pos = np.arange(SEQ, dtype=np.float64)[:, None]
    fill = pos * np.exp(
        -np.arange(0, D_MODEL, 2, dtype=np.float64) * math.log(10000.0) / D_MODEL
    )
    pe = np.zeros((SEQ, D_MODEL), dtype=np.float32)
    pe[:, 0::2] = np.sin(fill)
    pe[:, 1::2] = np.cos(fill)
    return pe


_PE = _pe_table()

_MESH = plsc.VectorSubcoreMesh(core_axis_name="c", subcore_axis_name="s")


@functools.partial(
    pl.kernel,
    out_type=jax.ShapeDtypeStruct((BATCH * SEQ, D_MODEL), jnp.float32),
    mesh=_MESH,
    scratch_types=[
        pltpu.VMEM((NCHUNK, CHUNK), jnp.int32),
        pltpu.VMEM((CHUNK, D_MODEL), jnp.float32),
        pltpu.VMEM((CHUNK, D_MODEL), jnp.float32),
        pltpu.SemaphoreType.DMA,
    ],
)
def _sc_embed(x_hbm, emb_hbm, pe_hbm, out_hbm, idx_v, rows_v, pe_v, gsem):
    wid = lax.axis_index("s") * 2 + lax.axis_index("c")
    base = wid * ROWS_PER_W
    pe_base = (wid % SEQ_PER_W) * ROWS_PER_W

    pltpu.sync_copy(x_hbm.at[wid], idx_v)

    def chunk_body(g, carry):
        row0 = base + g * CHUNK
        pos0 = pe_base + g * CHUNK
        pltpu.sync_copy(pe_hbm.at[pl.ds(pos0, CHUNK)], pe_v)
        pltpu.async_copy(emb_hbm.at[idx_v.at[g]], rows_v, gsem).wait()

        def add_row(r, carry2):
            for j in range(VPR):
                plsc.addupdate(
                    rows_v.at[r, pl.ds(j * LANES, LANES)],
                    pe_v[r, pl.ds(j * LANES, LANES)],
                )
            return carry2

        lax.fori_loop(0, CHUNK, add_row, 0)
        pltpu.sync_copy(rows_v, out_hbm.at[pl.ds(row0, CHUNK)])
        return carry

    lax.fori_loop(0, NCHUNK, chunk_body, 0)


def kernel(x, emb):
    x_r = x.reshape(NW, NCHUNK, CHUNK).astype(jnp.int32)
    pe = jnp.asarray(_PE)
    out = _sc_embed(x_r, emb, pe)
    return out.reshape(BATCH, SEQ, D_MODEL)


# speedup vs baseline: 1.3131x; 1.3131x over previous
import functools
import math

import jax
import jax.numpy as jnp
import numpy as np
from jax import lax
from jax.experimental import pallas as pl
from jax.experimental.pallas import tpu as pltpu
from jax.experimental.pallas import tpu_sc as plsc

D_MODEL = 1024
SEQ = 4096
BATCH = 4
LANES = 16

NCORE = 2
NSUB = 16
POS_PER_W = SEQ // (NCORE * NSUB)
CHUNK = 32
NPCHUNK = POS_PER_W // CHUNK
NSTEP = NPCHUNK * BATCH
VPR = D_MODEL // LANES


def _pe_table():
    pos = np.arange(SEQ, dtype=np.float64)[:, None]
    fill = pos * np.exp(
        -np.arange(0, D_MODEL, 2, dtype=np.float64) * math.log(10000.0) / D_MODEL
    )
    pe = np.zeros((SEQ, D_MODEL), dtype=np.float32)
    pe[:, 0::2] = np.sin(fill)
    pe[:, 1::2] = np.cos(fill)
    return pe


_PE = _pe_table()

_MESH = plsc.VectorSubcoreMesh(core_axis_name="c", subcore_axis_name="s")


@functools.partial(
    pl.kernel,
    out_type=jax.ShapeDtypeStruct((BATCH * SEQ, D_MODEL), jnp.float32),
    mesh=_MESH,
    scratch_types=[
        pltpu.VMEM((NPCHUNK, BATCH, CHUNK), jnp.int32),
        pltpu.VMEM((2, CHUNK, D_MODEL), jnp.float32),
        pltpu.VMEM((CHUNK, D_MODEL), jnp.float32),
        pltpu.SemaphoreType.DMA,
        pltpu.SemaphoreType.DMA,
        pltpu.SemaphoreType.DMA,
        pltpu.SemaphoreType.DMA,
    ],
)
def _sc_embed(x_hbm, emb_hbm, pe_hbm, out_hbm, idx_v, rows_v, pe_v, g0, g1, s0, s1):
    c = lax.axis_index("c")
    s = lax.axis_index("s")
    pos_base = c * (SEQ // NCORE) + s * POS_PER_W

    gsem = [g0, g1]
    ssem = [s0, s1]

    pltpu.sync_copy(x_hbm.at[c, s], idx_v)

    def gather(t, buf):
        p, b = divmod(t, BATCH)
        return pltpu.async_copy(
            emb_hbm.at[idx_v.at[p, b]], rows_v.at[buf], gsem[buf]
        )

    def store(t, buf):
        p, b = divmod(t, BATCH)
        row0 = b * SEQ + pos_base + p * CHUNK
        return pltpu.async_copy(
            rows_v.at[buf], out_hbm.at[pl.ds(row0, CHUNK)], ssem[buf]
        )

    gathers = [None, None]
    stores = [None, None]
    gathers[0] = gather(0, 0)

    for t in range(NSTEP):
        buf = t % 2
        if t % BATCH == 0:
            pos0 = pos_base + (t // BATCH) * CHUNK
            pltpu.sync_copy(pe_hbm.at[pl.ds(pos0, CHUNK)], pe_v)
        gathers[buf].wait()
        if t + 1 < NSTEP:
            if t >= 1:
                stores[1 - buf].wait()
            gathers[1 - buf] = gather(t + 1, 1 - buf)

        def add_row(r, carry):
            for j in range(VPR):
                plsc.addupdate(
                    rows_v.at[buf, r, pl.ds(j * LANES, LANES)],
                    pe_v[r, pl.ds(j * LANES, LANES)],
                )
            return carry

        lax.fori_loop(0, CHUNK, add_row, 0)
        stores[buf] = store(t, buf)

    stores[0].wait()
    stores[1].wait()


def kernel(x, emb):
    x_r = (
        x.astype(jnp.int32)
        .reshape(BATCH, NCORE, NSUB, NPCHUNK, CHUNK)
        .transpose(1, 2, 3, 0, 4)
    )
    pe = jnp.asarray(_PE)
    out = _sc_embed(x_r, emb, pe)
    return out.reshape(BATCH, SEQ, D_MODEL)
